# baseline (speedup 1.0000x reference)
import functools

import jax
import jax.numpy as jnp
from jax import lax
from jax.experimental import pallas as pl
from jax.experimental.pallas import tpu as pltpu
from jax.experimental.pallas import tpu_sc as plsc

N_FIELDS = 26
VOCAB = 100000
EMB_DIM = 16
BATCH = 16384
N_CONT = 13
BN_EPS = 1e-5

R_EMB = N_FIELDS * EMB_DIM
R_TOT = R_EMB + N_CONT
NW = 32
RPW = R_EMB // NW
HSPLIT = 50048
HREM = VOCAB - HSPLIT
E = 2048
NE = BATCH // E
NOB = 4


def _rsqrt_newton(x):
  seed = plsc.bitcast(
      jnp.int32(0x5F3759DF) - (plsc.bitcast(x, jnp.int32) >> 1), jnp.float32)
  y = seed
  for _ in range(4):
    y = y * (1.5 - 0.5 * x * y * y)
  return y


def _sc_encode(xT, T2, xcT, g16, b16):
  mesh = plsc.VectorSubcoreMesh(core_axis_name="c", subcore_axis_name="s")

  @functools.partial(
      pl.kernel,
      mesh=mesh,
      out_type=jax.ShapeDtypeStruct((R_TOT, BATCH), jnp.float32),
      scratch_types=[
          pltpu.VMEM((HSPLIT,), jnp.float32),
          pltpu.VMEM((HREM,), jnp.float32),
          pltpu.VMEM((BATCH,), jnp.int32),
          pltpu.VMEM((E,), jnp.float32),
          pltpu.VMEM((E,), jnp.float32),
          pltpu.VMEM((E,), jnp.float32),
          pltpu.VMEM((E,), jnp.float32),
          pltpu.VMEM((16,), jnp.float32),
          pltpu.VMEM((16,), jnp.float32),
          pltpu.SemaphoreType.DMA,
          pltpu.SemaphoreType.DMA,
          pltpu.SemaphoreType.DMA,
          pltpu.SemaphoreType.DMA,
          pltpu.SemaphoreType.DMA,
          pltpu.SemaphoreType.DMA,
      ],
      compiler_params=pltpu.CompilerParams(
          use_tc_tiling_on_sc=True, needs_layout_passes=False),
  )
  def k(xT_h, T2_h, xcT_h, g_h, b_h, out_h, h0b, h1b, idxb,
        ob0, ob1, ob2, ob3, gb, bb, asem, bsem, os0, os1, os2, os3):
    wid = lax.axis_index("s") * 2 + lax.axis_index("c")
    base = wid * RPW
    obufs = (ob0, ob1, ob2, ob3)
    osems = (os0, os1, os2, os3)

    @pl.when(wid >= NW - N_CONT)
    def _():
      f = wid - (NW - N_CONT)
      c = R_EMB + f
      pltpu.sync_copy(xcT_h.at[f], h0b.at[pl.ds(0, BATCH)])
      pltpu.sync_copy(g_h, gb)
      pltpu.sync_copy(b_h, bb)

      def acc(t, carry):
        s, q = carry
        v = h0b[pl.ds(t * 16, 16)]
        return s + v, q + v * v

      z = jnp.zeros((16,), jnp.float32)
      s, q = lax.fori_loop(0, BATCH // 16, acc, (z, z))
      mean = jnp.sum(s) * (1.0 / BATCH)
      var = jnp.sum(q) * (1.0 / BATCH) - mean * mean
      fv = jnp.full((16,), f, jnp.int32)
      gval = plsc.load_gather(gb, [fv])
      bval = plsc.load_gather(bb, [fv])
      rstd = _rsqrt_newton(jnp.full((16,), var + BN_EPS, jnp.float32))
      scale = gval * rstd
      shift = bval - jnp.full((16,), mean, jnp.float32) * scale

      def nchunk(kk, cc):
        def nvec(t, c2):
          v = h0b[pl.ds(kk * E + t * 16, 16)]
          ob0[pl.ds(t * 16, 16)] = v * scale + shift
          return c2

        lax.fori_loop(0, E // 16, nvec, 0, unroll=8)
        pltpu.sync_copy(ob0, out_h.at[c, pl.ds(kk * E, E)])
        return cc

      lax.fori_loop(0, NE, nchunk, 0)

    pltpu.make_async_copy(T2_h.at[base].at[pl.ds(0, HSPLIT)], h0b, asem).start()

    def row_body(j, prev_field):
      c = base + j
      i = c // EMB_DIM

      @pl.when(i != prev_field)
      def _():
        pltpu.sync_copy(xT_h.at[i], idxb)

      pltpu.make_async_copy(T2_h.at[c].at[pl.ds(0, HSPLIT)], h0b, asem).wait()
      pltpu.make_async_copy(T2_h.at[c].at[pl.ds(HSPLIT, HREM)], h1b, bsem).start()

      outs = {}

      def pass0(e):
        buf = obufs[e % NOB]

        @plsc.parallel_loop(0, E, step=16, unroll=8)
        def _g0(t, _e=e, _buf=buf):
          iv = idxb[pl.ds(_e * E + t, 16)]
          ivc = jnp.minimum(iv, HSPLIT - 1)
          _buf[pl.ds(t, 16)] = plsc.load_gather(h0b, [ivc])

      def pass1(e):
        buf = obufs[e % NOB]

        @plsc.parallel_loop(0, E, step=16, unroll=8)
        def _g1(t, _e=e, _buf=buf):
          iv = idxb[pl.ds(_e * E + t, 16)]
          iv2 = jnp.maximum(iv - HSPLIT, 0)
          g = plsc.load_gather(h1b, [iv2])
          prev = _buf[pl.ds(t, 16)]
          _buf[pl.ds(t, 16)] = jnp.where(iv >= HSPLIT, g, prev)

        cp = pltpu.make_async_copy(buf, out_h.at[c, pl.ds(e * E, E)],
                                   osems[e % NOB])
        cp.start()
        outs[e] = cp

      for e in range(4):
        pass0(e)
      pltpu.make_async_copy(T2_h.at[c].at[pl.ds(HSPLIT, HREM)], h1b, bsem).wait()
      for e in range(4):
        pass1(e)
      for e in range(4, NE):
        outs[e - 4].wait()
        pass0(e)

      @pl.when(j + 1 < RPW)
      def _():
        pltpu.make_async_copy(T2_h.at[c + 1].at[pl.ds(0, HSPLIT)], h0b, asem).start()

      for e in range(4, NE):
        pass1(e)
      for e in range(4, NE):
        outs[e].wait()
      return i

    lax.fori_loop(0, RPW, row_body, jnp.int32(-1))

  return k(xT, T2, xcT, g16, b16)


def kernel(x_cat, x_cont, tables, gamma, beta):
  xT = x_cat.astype(jnp.int32).T
  T2 = jnp.transpose(tables, (0, 2, 1)).reshape(R_EMB, VOCAB)
  xcT = x_cont.T
  g16 = jnp.pad(gamma, (0, 16 - N_CONT))
  b16 = jnp.pad(beta, (0, 16 - N_CONT))
  outT = _sc_encode(xT, T2, xcT, g16, b16)
  return outT.T


# speedup vs baseline: 1.1940x; 1.1940x over previous
import functools

import jax
import jax.numpy as jnp
from jax import lax
from jax.experimental import pallas as pl
from jax.experimental.pallas import tpu as pltpu
from jax.experimental.pallas import tpu_sc as plsc

N_FIELDS = 26
VOCAB = 100000
EMB_DIM = 16
BATCH = 16384
N_CONT = 13
BN_EPS = 1e-5

R_EMB = N_FIELDS * EMB_DIM
R_TOT = R_EMB + N_CONT
NW = 32
RPW = R_EMB // NW
CH = 4096
NCH = BATCH // CH


def _rsqrt_newton(x):
  seed = plsc.bitcast(
      jnp.int32(0x5F3759DF) - (plsc.bitcast(x, jnp.int32) >> 1), jnp.float32)
  y = seed
  for _ in range(4):
    y = y * (1.5 - 0.5 * x * y * y)
  return y


def _sc_encode(xT, T2, xcT, g16, b16):
  mesh = plsc.VectorSubcoreMesh(core_axis_name="c", subcore_axis_name="s")

  @functools.partial(
      pl.kernel,
      mesh=mesh,
      out_type=jax.ShapeDtypeStruct((R_TOT, BATCH), jnp.float32),
      scratch_types=[
          pltpu.VMEM((VOCAB,), jnp.float32),
          pltpu.VMEM((BATCH,), jnp.int32),
          pltpu.VMEM((CH,), jnp.float32),
          pltpu.VMEM((CH,), jnp.float32),
          pltpu.VMEM((16,), jnp.float32),
          pltpu.VMEM((16,), jnp.float32),
          pltpu.SemaphoreType.DMA,
          pltpu.SemaphoreType.DMA,
      ],
      compiler_params=pltpu.CompilerParams(
          use_tc_tiling_on_sc=True, needs_layout_passes=False),
  )
  def k(xT_h, T2_h, xcT_h, g_h, b_h, out_h, rowb, idxb, outb0, outb1, gb, bb,
        sem0, sem1):
    wid = lax.axis_index("s") * 2 + lax.axis_index("c")
    base = wid * RPW
    obufs = (outb0, outb1)
    osems = (sem0, sem1)

    def row_body(j, prev_field):
      c = base + j
      i = c // EMB_DIM

      @pl.when(i != prev_field)
      def _():
        pltpu.sync_copy(xT_h.at[i], idxb)

      pltpu.sync_copy(T2_h.at[c], rowb)

      writes = []
      for kk in range(NCH):
        buf = obufs[kk % 2]
        if kk >= 2:
          writes[kk - 2].wait()

        @plsc.parallel_loop(0, CH, step=16, unroll=8)
        def _gather(t, _kk=kk, _buf=buf):
          iv = idxb[pl.ds(_kk * CH + t, 16)]
          _buf[pl.ds(t, 16)] = plsc.load_gather(rowb, [iv])
        cp = pltpu.make_async_copy(buf, out_h.at[c, pl.ds(kk * CH, CH)],
                                   osems[kk % 2])
        cp.start()
        writes.append(cp)
      writes[NCH - 2].wait()
      writes[NCH - 1].wait()
      return i

    @pl.when(wid >= NW - N_CONT)
    def _():
      f = wid - (NW - N_CONT)
      c = R_EMB + f
      pltpu.sync_copy(xcT_h.at[f], rowb.at[pl.ds(0, BATCH)])
      pltpu.sync_copy(g_h, gb)
      pltpu.sync_copy(b_h, bb)

      def acc(t, carry):
        s, q = carry
        v = rowb[pl.ds(t * 16, 16)]
        return s + v, q + v * v

      z = jnp.zeros((16,), jnp.float32)
      s, q = lax.fori_loop(0, BATCH // 16, acc, (z, z))
      tot = jnp.sum(s)
      mean = tot * (1.0 / BATCH)
      var = jnp.sum(q) * (1.0 / BATCH) - mean * mean
      fv = jnp.full((16,), f, jnp.int32)
      gval = plsc.load_gather(gb, [fv])
      bval = plsc.load_gather(bb, [fv])
      rstd = _rsqrt_newton(jnp.full((16,), var + BN_EPS, jnp.float32))
      scale = gval * rstd
      shift = bval - jnp.full((16,), mean, jnp.float32) * scale

      def nchunk(kk, cc):
        def nvec(t, c2):
          v = rowb[pl.ds(kk * CH + t * 16, 16)]
          outb0[pl.ds(t * 16, 16)] = v * scale + shift
          return c2

        lax.fori_loop(0, CH // 16, nvec, 0, unroll=8)
        pltpu.sync_copy(outb0, out_h.at[c, pl.ds(kk * CH, CH)])
        return cc

      lax.fori_loop(0, NCH, nchunk, 0)

    lax.fori_loop(0, RPW, row_body, jnp.int32(-1))


  return k(xT, T2, xcT, g16, b16)


def kernel(x_cat, x_cont, tables, gamma, beta):
  xT = x_cat.astype(jnp.int32).T
  T2 = jnp.transpose(tables, (0, 2, 1)).reshape(R_EMB, VOCAB)
  xcT = x_cont.T
  g16 = jnp.pad(gamma, (0, 16 - N_CONT))
  b16 = jnp.pad(beta, (0, 16 - N_CONT))
  outT = _sc_encode(xT, T2, xcT, g16, b16)
  return outT.T


# speedup vs baseline: 1.1952x; 1.0010x over previous
import functools

import jax
import jax.numpy as jnp
from jax import lax
from jax.experimental import pallas as pl
from jax.experimental.pallas import tpu as pltpu
from jax.experimental.pallas import tpu_sc as plsc

N_FIELDS = 26
VOCAB = 100000
EMB_DIM = 16
BATCH = 16384
N_CONT = 13
BN_EPS = 1e-5

R_EMB = N_FIELDS * EMB_DIM
R_TOT = R_EMB + N_CONT
NW = 32
RPW = R_EMB // NW
CH = 4096
NCH = BATCH // CH


def _rsqrt_newton(x):
  seed = plsc.bitcast(
      jnp.int32(0x5F3759DF) - (plsc.bitcast(x, jnp.int32) >> 1), jnp.float32)
  y = seed
  for _ in range(4):
    y = y * (1.5 - 0.5 * x * y * y)
  return y


def _sc_encode(xT, T2, xcT, g16, b16):
  mesh = plsc.VectorSubcoreMesh(core_axis_name="c", subcore_axis_name="s")

  @functools.partial(
      pl.kernel,
      mesh=mesh,
      out_type=jax.ShapeDtypeStruct((R_TOT, BATCH), jnp.float32),
      scratch_types=[
          pltpu.VMEM((VOCAB,), jnp.float32),
          pltpu.VMEM((BATCH,), jnp.int32),
          pltpu.VMEM((CH,), jnp.float32),
          pltpu.VMEM((CH,), jnp.float32),
          pltpu.VMEM((16,), jnp.float32),
          pltpu.VMEM((16,), jnp.float32),
          pltpu.SemaphoreType.DMA,
          pltpu.SemaphoreType.DMA,
      ],
      compiler_params=pltpu.CompilerParams(
          use_tc_tiling_on_sc=True, needs_layout_passes=False),
  )
  def k(xT_h, T2_h, xcT_h, g_h, b_h, out_h, rowb, idxb, outb0, outb1, gb, bb,
        sem0, sem1):
    wid = lax.axis_index("s") * 2 + lax.axis_index("c")
    base = wid * RPW
    obufs = (outb0, outb1)
    osems = (sem0, sem1)

    def row_body(j, prev_field):
      c = base + j
      i = c // EMB_DIM

      @pl.when(i != prev_field)
      def _():
        pltpu.sync_copy(xT_h.at[i], idxb)

      pltpu.sync_copy(T2_h.at[c], rowb)

      writes = []
      for kk in range(NCH):
        buf = obufs[kk % 2]
        if kk >= 2:
          writes[kk - 2].wait()

        @plsc.parallel_loop(0, CH, step=16, unroll=16)
        def _gather(t, _kk=kk, _buf=buf):
          iv = idxb[pl.ds(_kk * CH + t, 16)]
          _buf[pl.ds(t, 16)] = plsc.load_gather(rowb, [iv])
        cp = pltpu.make_async_copy(buf, out_h.at[c, pl.ds(kk * CH, CH)],
                                   osems[kk % 2])
        cp.start()
        writes.append(cp)
      writes[NCH - 2].wait()
      writes[NCH - 1].wait()
      return i

    @pl.when(wid >= NW - N_CONT)
    def _():
      f = wid - (NW - N_CONT)
      c = R_EMB + f
      pltpu.sync_copy(xcT_h.at[f], rowb.at[pl.ds(0, BATCH)])
      pltpu.sync_copy(g_h, gb)
      pltpu.sync_copy(b_h, bb)

      def acc(t, carry):
        s, q = carry
        v = rowb[pl.ds(t * 16, 16)]
        return s + v, q + v * v

      z = jnp.zeros((16,), jnp.float32)
      s, q = lax.fori_loop(0, BATCH // 16, acc, (z, z))
      tot = jnp.sum(s)
      mean = tot * (1.0 / BATCH)
      var = jnp.sum(q) * (1.0 / BATCH) - mean * mean
      fv = jnp.full((16,), f, jnp.int32)
      gval = plsc.load_gather(gb, [fv])
      bval = plsc.load_gather(bb, [fv])
      rstd = _rsqrt_newton(jnp.full((16,), var + BN_EPS, jnp.float32))
      scale = gval * rstd
      shift = bval - jnp.full((16,), mean, jnp.float32) * scale

      def nchunk(kk, cc):
        def nvec(t, c2):
          v = rowb[pl.ds(kk * CH + t * 16, 16)]
          outb0[pl.ds(t * 16, 16)] = v * scale + shift
          return c2

        lax.fori_loop(0, CH // 16, nvec, 0, unroll=8)
        pltpu.sync_copy(outb0, out_h.at[c, pl.ds(kk * CH, CH)])
        return cc

      lax.fori_loop(0, NCH, nchunk, 0)

    lax.fori_loop(0, RPW, row_body, jnp.int32(-1))


  return k(xT, T2, xcT, g16, b16)


def kernel(x_cat, x_cont, tables, gamma, beta):
  xT = x_cat.astype(jnp.int32).T
  T2 = jnp.transpose(tables, (0, 2, 1)).reshape(R_EMB, VOCAB)
  xcT = x_cont.T
  g16 = jnp.pad(gamma, (0, 16 - N_CONT))
  b16 = jnp.pad(beta, (0, 16 - N_CONT))
  outT = _sc_encode(xT, T2, xcT, g16, b16)
  return outT.T
